# baseline (speedup 1.0000x reference)
import functools

import jax
import jax.numpy as jnp
from jax import lax
from jax.experimental import pallas as pl
from jax.experimental.pallas import tpu as pltpu
from jax.experimental.pallas import tpu_sc as plsc

N = 10000
D = 128
E = 320000
DE = 16
NPAD = 10240

NC = 2
NS = 16
NW = NC * NS
EPW = E // NW
CB = 80
NCH = EPW // CB
LPC = CB // 16
RPT = NPAD // NS

_mesh = plsc.VectorSubcoreMesh(core_axis_name="c", subcore_axis_name="s")


def _proj_body(x_ref, wdd_ref, wdt_ref, wtt_ref,
               asdd_ref, atdd_ref, asdt_ref, atdt_ref, astt_ref, attt_ref,
               ea_ref, b_ref,
               hdd_ref, hdt_ref, htt_ref,
               sdd_ref, tdd_ref, sdt_ref, tdt_ref, stt_ref, ttt_ref,
               et_ref):
    x = x_ref[...]
    et_ref[...] = jnp.dot(ea_ref[...], b_ref[...],
                          preferred_element_type=jnp.float32)
    wrefs = (wdd_ref, wdt_ref, wtt_ref)
    arefs = (asdd_ref, atdd_ref, asdt_ref, atdt_ref, astt_ref, attt_ref)
    hrefs = (hdd_ref, hdt_ref, htt_ref)
    srefs = (sdd_ref, tdd_ref, sdt_ref, tdt_ref, stt_ref, ttt_ref)
    for r in range(3):
        h = jnp.dot(x, wrefs[r][...], preferred_element_type=jnp.float32)
        hrefs[r][...] = h
        h3 = h.reshape(8, 128, D)
        for p in range(2):
            a = arefs[2 * r + p][0, :]
            srefs[2 * r + p][...] = jnp.sum(h3 * a[None, None, :], axis=2)


def _project(x, W_dd, W_dt, W_tt, a6, ea2, B):
    blk = 1024
    eblk = E // 8 // (NPAD // blk)
    grid = (NPAD // blk,)
    wspec = pl.BlockSpec((D, D), lambda i: (0, 0))
    aspec = pl.BlockSpec((1, D), lambda i: (0, 0))
    hspec = pl.BlockSpec((blk, D), lambda i: (i, 0))
    sspec = pl.BlockSpec((8, D), lambda i: (i, 0))
    espec = pl.BlockSpec((eblk, 128), lambda i: (i, 0))
    return pl.pallas_call(
        _proj_body,
        grid=grid,
        in_specs=[hspec, wspec, wspec, wspec] + [aspec] * 6
        + [espec, pl.BlockSpec((128, 128), lambda i: (0, 0))],
        out_specs=[hspec] * 3 + [sspec] * 6 + [espec],
        out_shape=[jax.ShapeDtypeStruct((NPAD, D), jnp.float32)] * 3
        + [jax.ShapeDtypeStruct((NPAD // 128, 128), jnp.float32)] * 6
        + [jax.ShapeDtypeStruct((E // 8, 128), jnp.float32)],
    )(x, W_dd, W_dt, W_tt, *a6, ea2, B)


@functools.partial(
    pl.kernel,
    out_type=[jax.ShapeDtypeStruct((E,), jnp.float32)] * 3
    + [jax.ShapeDtypeStruct((NC, NPAD), jnp.float32)] * 3,
    mesh=_mesh,
    scratch_types=[
        pltpu.VMEM((NPAD,), jnp.float32),
        pltpu.VMEM((NPAD,), jnp.float32),
        pltpu.VMEM((EPW,), jnp.int32),
        pltpu.VMEM((NCH, CB), jnp.int32),
        pltpu.VMEM((EPW,), jnp.float32),
        pltpu.VMEM((EPW,), jnp.float32),
        pltpu.VMEM_SHARED((NPAD,), jnp.float32),
        pltpu.VMEM_SHARED((NPAD,), jnp.float32),
        pltpu.VMEM_SHARED((NPAD,), jnp.float32),
    ],
    compiler_params=pltpu.CompilerParams(needs_layout_passes=False),
)
def _sc_edge_logits(s_dd, t_dd, s_dt, t_dt, s_tt, t_tt,
                    src_dd, src_dt, src_tt, dst2_dd, dst2_dt, dst2_tt,
                    eatt_hbm,
                    ex_dd, ex_dt, ex_tt, den_dd, den_dt, den_tt,
                    s_v, t_v, src_v, dst_v, ea_v, ex_v, dsh0, dsh1, dsh2):
    cid = lax.axis_index("c")
    sid = lax.axis_index("s")
    wid = cid * NS + sid
    base = pl.multiple_of(wid * EPW, 8)
    rbase = pl.multiple_of(sid * RPT, 8)
    dshs = (dsh0, dsh1, dsh2)
    s_hbms = (s_dd, s_dt, s_tt)
    t_hbms = (t_dd, t_dt, t_tt)
    src_hbms = (src_dd, src_dt, src_tt)
    dst_hbms = (dst2_dd, dst2_dt, dst2_tt)
    ex_hbms = (ex_dd, ex_dt, ex_tt)
    den_hbms = (den_dd, den_dt, den_tt)

    for m in range(RPT // 16):
        ex_v[pl.ds(m * 16, 16)] = jnp.zeros((16,), jnp.float32)
    for r in range(3):
        pltpu.sync_copy(ex_v.at[pl.ds(0, RPT)], dshs[r].at[pl.ds(rbase, RPT)])
    plsc.subcore_barrier()

    for r in range(3):
        pltpu.sync_copy(s_hbms[r], s_v)
        pltpu.sync_copy(t_hbms[r], t_v)
        pltpu.sync_copy(src_hbms[r].at[pl.ds(base, EPW)], src_v)
        pltpu.sync_copy(dst_hbms[r].at[wid], dst_v)
        if r == 0:
            pltpu.sync_copy(eatt_hbm.at[pl.ds(base, EPW)], ea_v)

        def chunk_body(ch, carry, r=r):
            for k in range(LPC):
                off = ch * CB + k * 16
                si = src_v[pl.ds(off, 16)]
                ti = dst_v[ch, pl.ds(k * 16, 16)]
                e = plsc.load_gather(s_v, [si]) + plsc.load_gather(t_v, [ti])
                if r == 0:
                    e = e + ea_v[pl.ds(off, 16)]
                e = jnp.where(e >= 0.0, e, 0.2 * e)
                ex_v[pl.ds(off, 16)] = jnp.exp(e)
            pltpu.sync_copy(ex_v.at[pl.ds(ch * CB, CB)],
                            dshs[r].at[dst_v.at[ch]], add=True)
            return carry

        lax.fori_loop(0, NCH, chunk_body, 0)
        pltpu.sync_copy(ex_v, ex_hbms[r].at[pl.ds(base, EPW)])

    plsc.subcore_barrier()
    for r in range(3):
        pltpu.sync_copy(dshs[r].at[pl.ds(rbase, RPT)],
                        den_hbms[r].at[cid, pl.ds(rbase, RPT)])


def _den_combine_body(d0_ref, d1_ref, d2_ref, o0_ref, o1_ref, o2_ref):
    for d_ref, o_ref in ((d0_ref, o0_ref), (d1_ref, o1_ref), (d2_ref, o2_ref)):
        o_ref[...] = d_ref[0] + d_ref[1] + jnp.float32(1e-16)


def _den_combine(den3):
    ispec = pl.BlockSpec((NC, 8, 128), lambda i: (0, i, 0))
    ospec = pl.BlockSpec((8, 128), lambda i: (i, 0))
    return pl.pallas_call(
        _den_combine_body,
        grid=(NPAD // 1024,),
        in_specs=[ispec] * 3,
        out_specs=[ospec] * 3,
        out_shape=[jax.ShapeDtypeStruct((NPAD // 128, 128), jnp.float32)] * 3,
    )(*[d.reshape(NC, NPAD // 128, 128) for d in den3])


CB2 = 40
NCH2 = EPW // CB2
_GRP = ((0, 0), (16, 0), (24, 8))


@functools.partial(
    pl.kernel,
    out_type=jax.ShapeDtypeStruct((NC, NPAD, D), jnp.float32),
    mesh=_mesh,
    scratch_types=[
        pltpu.VMEM((EPW,), jnp.int32),
        pltpu.VMEM((EPW,), jnp.float32),
        pltpu.VMEM((4, CB2), jnp.int32),
        pltpu.VMEM((2 * CB2,), jnp.float32),
        pltpu.VMEM((CB2, D), jnp.float32),
        pltpu.VMEM((CB2, D), jnp.float32),
        pltpu.VMEM((CB2, D), jnp.float32),
        pltpu.VMEM((CB2, D), jnp.float32),
        pltpu.SemaphoreType.DMA,
        pltpu.SemaphoreType.DMA,
        pltpu.SemaphoreType.DMA,
        pltpu.SemaphoreType.DMA,
        pltpu.VMEM_SHARED((NPAD, D), jnp.float32),
    ],
    compiler_params=pltpu.CompilerParams(needs_layout_passes=False),
)
def _sc_aggregate(h_dd, h_dt, h_tt, ex_dd, ex_dt, ex_tt,
                  src2_dd, src2_dt, src2_tt, dst2_dd, dst2_dt, dst2_tt,
                  den_dd, den_dt, den_tt,
                  out_hbm,
                  src_v, den_v, dst_r, att_r, g0, g1, s0, s1,
                  gsem_a, gsem_b, ssem_a, ssem_b, accum):
    cid = lax.axis_index("c")
    sid = lax.axis_index("s")
    wid = cid * NS + sid
    base = pl.multiple_of(wid * EPW, 8)
    h_hbms = (h_dd, h_dt, h_tt)
    att_hbms = (ex_dd, ex_dt, ex_tt)
    den_hbms = (den_dd, den_dt, den_tt)
    src_hbms = (src2_dd, src2_dt, src2_tt)
    dst_hbms = (dst2_dd, dst2_dt, dst2_tt)
    gbuf = (g0, g1)
    sbuf = (s0, s1)
    gsems = (gsem_a, gsem_b)
    ssems = (ssem_a, ssem_b)

    for i in range(CB2):
        for u in range(D // 16):
            s0[i, pl.ds(u * 16, 16)] = jnp.zeros((16,), jnp.float32)
    for q in range(RPT // CB2):
        st = pl.multiple_of(sid * RPT + q * CB2, 8)
        pltpu.sync_copy(s0, accum.at[pl.ds(st, CB2)])
    plsc.subcore_barrier()

    for r in range(3):
        pltpu.sync_copy(src_hbms[r].at[pl.ds(base, EPW)], src_v)
        pltpu.sync_copy(den_hbms[r].at[pl.ds(0, EPW)], den_v)

        def prefetch(ch, k4, r=r):
            gk = k4 % 2
            cbase = pl.multiple_of(base + ch * CB2, 8)
            pltpu.async_copy(att_hbms[r].at[pl.ds(cbase, CB2)],
                             att_r.at[pl.ds(gk * CB2, CB2)], gsems[gk])
            pltpu.async_copy(dst_hbms[r].at[wid * NCH2 + ch],
                             dst_r.at[k4], gsems[gk])
            pltpu.async_copy(h_hbms[r].at[src_v.at[pl.ds(ch * CB2, CB2)]],
                             gbuf[gk], gsems[gk])

        def wait_scatter(gk, k4):
            pltpu.make_async_copy(sbuf[gk], accum.at[dst_r.at[k4]],
                                  ssems[gk]).wait()

        def step(ch, k4, swait_pred, pre_ch, r=r):
            gk = k4 % 2
            cbase = pl.multiple_of(base + ch * CB2, 8)
            pltpu.make_async_copy(att_hbms[r].at[pl.ds(cbase, CB2)],
                                  att_r.at[pl.ds(gk * CB2, CB2)],
                                  gsems[gk]).wait()
            pltpu.make_async_copy(dst_hbms[r].at[wid * NCH2 + ch],
                                  dst_r.at[k4], gsems[gk]).wait()
            pltpu.make_async_copy(h_hbms[r].at[src_v.at[pl.ds(ch * CB2, CB2)]],
                                  gbuf[gk], gsems[gk]).wait()
            if swait_pred is None:
                wait_scatter(gk, (k4 + 2) % 4)
            elif swait_pred is not False:
                @pl.when(swait_pred)
                def _():
                    wait_scatter(gk, (k4 + 2) % 4)
            for off, j0 in _GRP:
                dstv = dst_r[k4, pl.ds(off, 16)]
                den16 = plsc.load_gather(den_v, [dstv])
                av16 = att_r[pl.ds(gk * CB2 + off, 16)] / den16
                for j in range(j0, 16):
                    i = off + j
                    for u in range(D // 16):
                        sl = pl.ds(u * 16, 16)
                        sbuf[gk][i, sl] = gbuf[gk][i, sl] * av16[j]
            pltpu.async_copy(sbuf[gk], accum.at[dst_r.at[k4]],
                             ssems[gk], add=True)
            if pre_ch is not None:
                pre, pred = pre_ch
                if pred is None:
                    prefetch(pre, (k4 + 2) % 4)
                else:
                    @pl.when(pred)
                    def _():
                        prefetch(pre, (k4 + 2) % 4)

        prefetch(0, 0)
        prefetch(1, 1)

        def quad(q, cc):
            c0 = 4 * q
            step(c0 + 0, 0, q > 0, (c0 + 2, None))
            step(c0 + 1, 1, q > 0, (c0 + 3, None))
            step(c0 + 2, 2, None, (c0 + 4, None))
            step(c0 + 3, 3, None, (c0 + 5, None))
            return cc

        lax.fori_loop(0, (NCH2 - 2) // 4, quad, 0)
        step(NCH2 - 2, 0, None, None)
        step(NCH2 - 1, 1, None, None)
        wait_scatter(0, 0)
        wait_scatter(1, 1)

    plsc.subcore_barrier()
    for q in range(RPT // CB2):
        st = pl.multiple_of(sid * RPT + q * CB2, 8)
        pltpu.sync_copy(accum.at[pl.ds(st, CB2)],
                        out_hbm.at[cid, pl.ds(st, CB2)])


def _combine_body(p0_ref, p1_ref, o_ref):
    o_ref[...] = (p0_ref[...] + p1_ref[...]) * jnp.float32(1.0 / 3.0)


def _combine(p0, p1):
    blk = 1024
    spec = pl.BlockSpec((blk, D), lambda i: (i, 0))
    return pl.pallas_call(
        _combine_body,
        grid=(NPAD // blk,),
        in_specs=[spec, spec],
        out_specs=spec,
        out_shape=jax.ShapeDtypeStruct((NPAD, D), jnp.float32),
    )(p0, p1)


def kernel(all_nodes, edge_index_dd, edge_index_dt, edge_index_tt, edge_attr_dd,
           W_dd, a_src_dd, a_dst_dd, a_edge_dd,
           W_dt, a_src_dt, a_dst_dt,
           W_tt, a_src_tt, a_dst_tt):
    x = jnp.concatenate(
        [all_nodes, jnp.zeros((NPAD - N, D), jnp.float32)], axis=0)
    a6 = [a.reshape(1, D) for a in
          (a_src_dd, a_dst_dd, a_src_dt, a_dst_dt, a_src_tt, a_dst_tt)]
    ea2 = edge_attr_dd.reshape(E // 8, 8 * DE)
    rows = ((jnp.arange(8) * DE)[:, None] + jnp.arange(DE)[None, :]).reshape(-1)
    cols = jnp.repeat(jnp.arange(8), DE)
    B = jnp.zeros((8 * DE, 128), jnp.float32).at[rows, cols].set(
        jnp.tile(a_edge_dd, 8))
    (h_dd, h_dt, h_tt,
     s2_dd, t2_dd, s2_dt, t2_dt, s2_tt, t2_tt, et_pack) = _project(
        x, W_dd, W_dt, W_tt, a6, ea2, B)
    eatt = et_pack[:, :8].reshape(E)
    s_dd, t_dd = s2_dd.reshape(NPAD), t2_dd.reshape(NPAD)
    s_dt, t_dt = s2_dt.reshape(NPAD), t2_dt.reshape(NPAD)
    s_tt, t_tt = s2_tt.reshape(NPAD), t2_tt.reshape(NPAD)

    src_dd, dst_dd = edge_index_dd[0], edge_index_dd[1]
    src_dt, dst_dt = edge_index_dt[0], edge_index_dt[1]
    src_tt, dst_tt = edge_index_tt[0], edge_index_tt[1]
    d2 = lambda a: a.reshape(NW, NCH, CB)

    (ex_dd, ex_dt, ex_tt, denp_dd, denp_dt, denp_tt) = _sc_edge_logits(
        s_dd, t_dd, s_dt, t_dt, s_tt, t_tt,
        src_dd, src_dt, src_tt, d2(dst_dd), d2(dst_dt), d2(dst_tt), eatt)

    den_dd, den_dt, den_tt = [d.reshape(NPAD) for d in
                              _den_combine((denp_dd, denp_dt, denp_tt))]

    dflat = lambda a: a.reshape(NW * NCH2, CB2)
    out_parts = _sc_aggregate(
        h_dd, h_dt, h_tt, ex_dd, ex_dt, ex_tt,
        src_dd, src_dt, src_tt,
        dflat(dst_dd), dflat(dst_dt), dflat(dst_tt),
        den_dd, den_dt, den_tt)

    out = _combine(out_parts[0], out_parts[1])
    return out[:N]


# speedup vs baseline: 1.0913x; 1.0913x over previous
import functools

import jax
import jax.numpy as jnp
from jax import lax
from jax.experimental import pallas as pl
from jax.experimental.pallas import tpu as pltpu
from jax.experimental.pallas import tpu_sc as plsc

N = 10000
D = 128
E = 320000
DE = 16
NPAD = 10240

NC = 2
NS = 16
NW = NC * NS
EPW = E // NW
CB = 80
NCH = EPW // CB
LPC = CB // 16
RPT = NPAD // NS

_mesh = plsc.VectorSubcoreMesh(core_axis_name="c", subcore_axis_name="s")


def _proj_body(x_ref, wdd_ref, wdt_ref, wtt_ref,
               asdd_ref, atdd_ref, asdt_ref, atdt_ref, astt_ref, attt_ref,
               ea_ref, b_ref,
               hdd_ref, hdt_ref, htt_ref,
               sdd_ref, tdd_ref, sdt_ref, tdt_ref, stt_ref, ttt_ref,
               et_ref):
    x = x_ref[...]
    et_ref[...] = jnp.dot(ea_ref[...], b_ref[...],
                          preferred_element_type=jnp.float32)
    wrefs = (wdd_ref, wdt_ref, wtt_ref)
    arefs = (asdd_ref, atdd_ref, asdt_ref, atdt_ref, astt_ref, attt_ref)
    hrefs = (hdd_ref, hdt_ref, htt_ref)
    srefs = (sdd_ref, tdd_ref, sdt_ref, tdt_ref, stt_ref, ttt_ref)
    for r in range(3):
        h = jnp.dot(x, wrefs[r][...], preferred_element_type=jnp.float32)
        hrefs[r][...] = h
        h3 = h.reshape(8, 128, D)
        for p in range(2):
            a = arefs[2 * r + p][0, :]
            srefs[2 * r + p][...] = jnp.sum(h3 * a[None, None, :], axis=2)


def _project(x, W_dd, W_dt, W_tt, a6, ea2, B):
    blk = 1024
    eblk = E // 8 // (NPAD // blk)
    grid = (NPAD // blk,)
    wspec = pl.BlockSpec((D, D), lambda i: (0, 0))
    aspec = pl.BlockSpec((1, D), lambda i: (0, 0))
    hspec = pl.BlockSpec((blk, D), lambda i: (i, 0))
    sspec = pl.BlockSpec((8, D), lambda i: (i, 0))
    espec = pl.BlockSpec((eblk, 128), lambda i: (i, 0))
    return pl.pallas_call(
        _proj_body,
        grid=grid,
        in_specs=[hspec, wspec, wspec, wspec] + [aspec] * 6
        + [espec, pl.BlockSpec((128, 128), lambda i: (0, 0))],
        out_specs=[hspec] * 3 + [sspec] * 6 + [espec],
        out_shape=[jax.ShapeDtypeStruct((NPAD, D), jnp.float32)] * 3
        + [jax.ShapeDtypeStruct((NPAD // 128, 128), jnp.float32)] * 6
        + [jax.ShapeDtypeStruct((E // 8, 128), jnp.float32)],
    )(x, W_dd, W_dt, W_tt, *a6, ea2, B)


@functools.partial(
    pl.kernel,
    out_type=[jax.ShapeDtypeStruct((E,), jnp.float32)] * 3
    + [jax.ShapeDtypeStruct((NC, NPAD), jnp.float32)] * 3,
    mesh=_mesh,
    scratch_types=[
        pltpu.VMEM((NPAD,), jnp.float32),
        pltpu.VMEM((NPAD,), jnp.float32),
        pltpu.VMEM((EPW,), jnp.int32),
        pltpu.VMEM((NCH, CB), jnp.int32),
        pltpu.VMEM((EPW,), jnp.float32),
        pltpu.VMEM((EPW,), jnp.float32),
        pltpu.VMEM_SHARED((NPAD,), jnp.float32),
        pltpu.VMEM_SHARED((NPAD,), jnp.float32),
        pltpu.VMEM_SHARED((NPAD,), jnp.float32),
    ],
    compiler_params=pltpu.CompilerParams(needs_layout_passes=False),
)
def _sc_edge_logits(s_dd, t_dd, s_dt, t_dt, s_tt, t_tt,
                    src_dd, src_dt, src_tt, dst2_dd, dst2_dt, dst2_tt,
                    eatt_hbm,
                    ex_dd, ex_dt, ex_tt, den_dd, den_dt, den_tt,
                    s_v, t_v, src_v, dst_v, ea_v, ex_v, dsh0, dsh1, dsh2):
    cid = lax.axis_index("c")
    sid = lax.axis_index("s")
    wid = cid * NS + sid
    base = pl.multiple_of(wid * EPW, 8)
    rbase = pl.multiple_of(sid * RPT, 8)
    dshs = (dsh0, dsh1, dsh2)
    s_hbms = (s_dd, s_dt, s_tt)
    t_hbms = (t_dd, t_dt, t_tt)
    src_hbms = (src_dd, src_dt, src_tt)
    dst_hbms = (dst2_dd, dst2_dt, dst2_tt)
    ex_hbms = (ex_dd, ex_dt, ex_tt)
    den_hbms = (den_dd, den_dt, den_tt)

    for m in range(RPT // 16):
        ex_v[pl.ds(m * 16, 16)] = jnp.zeros((16,), jnp.float32)
    for r in range(3):
        pltpu.sync_copy(ex_v.at[pl.ds(0, RPT)], dshs[r].at[pl.ds(rbase, RPT)])
    plsc.subcore_barrier()

    for r in range(3):
        pltpu.sync_copy(s_hbms[r], s_v)
        pltpu.sync_copy(t_hbms[r], t_v)
        pltpu.sync_copy(src_hbms[r].at[pl.ds(base, EPW)], src_v)
        pltpu.sync_copy(dst_hbms[r].at[wid], dst_v)
        if r == 0:
            pltpu.sync_copy(eatt_hbm.at[pl.ds(base, EPW)], ea_v)

        def chunk_body(ch, carry, r=r):
            for k in range(LPC):
                off = ch * CB + k * 16
                si = src_v[pl.ds(off, 16)]
                ti = dst_v[ch, pl.ds(k * 16, 16)]
                e = plsc.load_gather(s_v, [si]) + plsc.load_gather(t_v, [ti])
                if r == 0:
                    e = e + ea_v[pl.ds(off, 16)]
                e = jnp.where(e >= 0.0, e, 0.2 * e)
                ex_v[pl.ds(off, 16)] = jnp.exp(e)
            pltpu.sync_copy(ex_v.at[pl.ds(ch * CB, CB)],
                            dshs[r].at[dst_v.at[ch]], add=True)
            return carry

        lax.fori_loop(0, NCH, chunk_body, 0)
        pltpu.sync_copy(ex_v, ex_hbms[r].at[pl.ds(base, EPW)])

    plsc.subcore_barrier()
    for r in range(3):
        pltpu.sync_copy(dshs[r].at[pl.ds(rbase, RPT)],
                        den_hbms[r].at[cid, pl.ds(rbase, RPT)])


def _den_combine_body(d0_ref, d1_ref, d2_ref, o0_ref, o1_ref, o2_ref):
    for d_ref, o_ref in ((d0_ref, o0_ref), (d1_ref, o1_ref), (d2_ref, o2_ref)):
        o_ref[...] = d_ref[0] + d_ref[1] + jnp.float32(1e-16)


def _den_combine(den3):
    ispec = pl.BlockSpec((NC, 8, 128), lambda i: (0, i, 0))
    ospec = pl.BlockSpec((8, 128), lambda i: (i, 0))
    return pl.pallas_call(
        _den_combine_body,
        grid=(NPAD // 1024,),
        in_specs=[ispec] * 3,
        out_specs=[ospec] * 3,
        out_shape=[jax.ShapeDtypeStruct((NPAD // 128, 128), jnp.float32)] * 3,
    )(*[d.reshape(NC, NPAD // 128, 128) for d in den3])


CB2 = 40
NCH2 = EPW // CB2
_GRP = ((0, 0), (16, 0), (24, 8))


@functools.partial(
    pl.kernel,
    out_type=jax.ShapeDtypeStruct((NC, 3, NPAD, D), jnp.float32),
    mesh=_mesh,
    scratch_types=[
        pltpu.VMEM((EPW,), jnp.int32),
        pltpu.VMEM((4, CB2), jnp.int32),
        pltpu.VMEM((2 * CB2,), jnp.float32),
        pltpu.VMEM((CB2, D), jnp.float32),
        pltpu.VMEM((CB2, D), jnp.float32),
        pltpu.VMEM((CB2, D), jnp.float32),
        pltpu.VMEM((CB2, D), jnp.float32),
        pltpu.SemaphoreType.DMA,
        pltpu.SemaphoreType.DMA,
        pltpu.SemaphoreType.DMA,
        pltpu.SemaphoreType.DMA,
        pltpu.VMEM_SHARED((NPAD, D), jnp.float32),
    ],
    compiler_params=pltpu.CompilerParams(needs_layout_passes=False),
)
def _sc_aggregate(h_dd, h_dt, h_tt, ex_dd, ex_dt, ex_tt,
                  src2_dd, src2_dt, src2_tt, dst2_dd, dst2_dt, dst2_tt,
                  out_hbm,
                  src_v, dst_r, att_r, g0, g1, s0, s1,
                  gsem_a, gsem_b, ssem_a, ssem_b, accum):
    cid = lax.axis_index("c")
    sid = lax.axis_index("s")
    wid = cid * NS + sid
    base = pl.multiple_of(wid * EPW, 8)
    h_hbms = (h_dd, h_dt, h_tt)
    att_hbms = (ex_dd, ex_dt, ex_tt)
    src_hbms = (src2_dd, src2_dt, src2_tt)
    dst_hbms = (dst2_dd, dst2_dt, dst2_tt)
    gbuf = (g0, g1)
    sbuf = (s0, s1)
    gsems = (gsem_a, gsem_b)
    ssems = (ssem_a, ssem_b)

    for i in range(CB2):
        for u in range(D // 16):
            s0[i, pl.ds(u * 16, 16)] = jnp.zeros((16,), jnp.float32)
    for q in range(RPT // CB2):
        st = pl.multiple_of(sid * RPT + q * CB2, 8)
        pltpu.sync_copy(s0, accum.at[pl.ds(st, CB2)])
    plsc.subcore_barrier()

    for r in range(3):
        pltpu.sync_copy(src_hbms[r].at[pl.ds(base, EPW)], src_v)

        def prefetch(ch, k4, r=r):
            gk = k4 % 2
            cbase = pl.multiple_of(base + ch * CB2, 8)
            pltpu.async_copy(att_hbms[r].at[pl.ds(cbase, CB2)],
                             att_r.at[pl.ds(gk * CB2, CB2)], gsems[gk])
            pltpu.async_copy(dst_hbms[r].at[wid * NCH2 + ch],
                             dst_r.at[k4], gsems[gk])
            pltpu.async_copy(h_hbms[r].at[src_v.at[pl.ds(ch * CB2, CB2)]],
                             gbuf[gk], gsems[gk])

        def wait_scatter(gk, k4):
            pltpu.make_async_copy(sbuf[gk], accum.at[dst_r.at[k4]],
                                  ssems[gk]).wait()

        def step(ch, k4, swait_pred, pre_ch, r=r):
            gk = k4 % 2
            cbase = pl.multiple_of(base + ch * CB2, 8)
            pltpu.make_async_copy(att_hbms[r].at[pl.ds(cbase, CB2)],
                                  att_r.at[pl.ds(gk * CB2, CB2)],
                                  gsems[gk]).wait()
            pltpu.make_async_copy(dst_hbms[r].at[wid * NCH2 + ch],
                                  dst_r.at[k4], gsems[gk]).wait()
            pltpu.make_async_copy(h_hbms[r].at[src_v.at[pl.ds(ch * CB2, CB2)]],
                                  gbuf[gk], gsems[gk]).wait()
            if swait_pred is None:
                wait_scatter(gk, (k4 + 2) % 4)
            elif swait_pred is not False:
                @pl.when(swait_pred)
                def _():
                    wait_scatter(gk, (k4 + 2) % 4)
            for off, j0 in _GRP:
                av16 = att_r[pl.ds(gk * CB2 + off, 16)]
                for j in range(j0, 16):
                    i = off + j
                    for u in range(D // 16):
                        sl = pl.ds(u * 16, 16)
                        sbuf[gk][i, sl] = gbuf[gk][i, sl] * av16[j]
            pltpu.async_copy(sbuf[gk], accum.at[dst_r.at[k4]],
                             ssems[gk], add=True)
            if pre_ch is not None:
                pre, pred = pre_ch
                if pred is None:
                    prefetch(pre, (k4 + 2) % 4)
                else:
                    @pl.when(pred)
                    def _():
                        prefetch(pre, (k4 + 2) % 4)

        prefetch(0, 0)
        prefetch(1, 1)

        def quad(q, cc):
            c0 = 4 * q
            step(c0 + 0, 0, q > 0, (c0 + 2, None))
            step(c0 + 1, 1, q > 0, (c0 + 3, None))
            step(c0 + 2, 2, None, (c0 + 4, None))
            step(c0 + 3, 3, None, (c0 + 5, None))
            return cc

        lax.fori_loop(0, (NCH2 - 2) // 4, quad, 0)
        step(NCH2 - 2, 0, None, None)
        step(NCH2 - 1, 1, None, None)
        wait_scatter(0, 0)
        wait_scatter(1, 1)

        plsc.subcore_barrier()
        for q in range(RPT // CB2):
            st = pl.multiple_of(sid * RPT + q * CB2, 8)
            pltpu.sync_copy(accum.at[pl.ds(st, CB2)],
                            out_hbm.at[cid, r, pl.ds(st, CB2)])
        if r < 2:
            for i in range(CB2):
                for u in range(D // 16):
                    s0[i, pl.ds(u * 16, 16)] = jnp.zeros((16,), jnp.float32)
            for q in range(RPT // CB2):
                st = pl.multiple_of(sid * RPT + q * CB2, 8)
                pltpu.sync_copy(s0, accum.at[pl.ds(st, CB2)])
            plsc.subcore_barrier()


def _combine_body(p00, p01, p02, p10, p11, p12, d0, d1, d2, o_ref):
    acc = (p00[...] + p10[...]) / d0[...]
    acc = acc + (p01[...] + p11[...]) / d1[...]
    acc = acc + (p02[...] + p12[...]) / d2[...]
    o_ref[...] = acc * jnp.float32(1.0 / 3.0)


def _combine(parts, dens):
    blk = 1024
    spec = pl.BlockSpec((blk, D), lambda i: (i, 0))
    return pl.pallas_call(
        _combine_body,
        grid=(NPAD // blk,),
        in_specs=[spec] * 9,
        out_specs=spec,
        out_shape=jax.ShapeDtypeStruct((NPAD, D), jnp.float32),
    )(*parts, *dens)


def kernel(all_nodes, edge_index_dd, edge_index_dt, edge_index_tt, edge_attr_dd,
           W_dd, a_src_dd, a_dst_dd, a_edge_dd,
           W_dt, a_src_dt, a_dst_dt,
           W_tt, a_src_tt, a_dst_tt):
    x = jnp.concatenate(
        [all_nodes, jnp.zeros((NPAD - N, D), jnp.float32)], axis=0)
    a6 = [a.reshape(1, D) for a in
          (a_src_dd, a_dst_dd, a_src_dt, a_dst_dt, a_src_tt, a_dst_tt)]
    ea2 = edge_attr_dd.reshape(E // 8, 8 * DE)
    rows = ((jnp.arange(8) * DE)[:, None] + jnp.arange(DE)[None, :]).reshape(-1)
    cols = jnp.repeat(jnp.arange(8), DE)
    B = jnp.zeros((8 * DE, 128), jnp.float32).at[rows, cols].set(
        jnp.tile(a_edge_dd, 8))
    (h_dd, h_dt, h_tt,
     s2_dd, t2_dd, s2_dt, t2_dt, s2_tt, t2_tt, et_pack) = _project(
        x, W_dd, W_dt, W_tt, a6, ea2, B)
    eatt = et_pack[:, :8].reshape(E)
    s_dd, t_dd = s2_dd.reshape(NPAD), t2_dd.reshape(NPAD)
    s_dt, t_dt = s2_dt.reshape(NPAD), t2_dt.reshape(NPAD)
    s_tt, t_tt = s2_tt.reshape(NPAD), t2_tt.reshape(NPAD)

    src_dd, dst_dd = edge_index_dd[0], edge_index_dd[1]
    src_dt, dst_dt = edge_index_dt[0], edge_index_dt[1]
    src_tt, dst_tt = edge_index_tt[0], edge_index_tt[1]
    d2 = lambda a: a.reshape(NW, NCH, CB)

    (ex_dd, ex_dt, ex_tt, denp_dd, denp_dt, denp_tt) = _sc_edge_logits(
        s_dd, t_dd, s_dt, t_dt, s_tt, t_tt,
        src_dd, src_dt, src_tt, d2(dst_dd), d2(dst_dt), d2(dst_tt), eatt)

    dens = [jnp.broadcast_to(d.reshape(NPAD, 1), (NPAD, D)) for d in
            _den_combine((denp_dd, denp_dt, denp_tt))]

    dflat = lambda a: a.reshape(NW * NCH2, CB2)
    out_parts = _sc_aggregate(
        h_dd, h_dt, h_tt, ex_dd, ex_dt, ex_tt,
        src_dd, src_dt, src_tt,
        dflat(dst_dd), dflat(dst_dt), dflat(dst_tt))

    parts = [out_parts[c, r] for c in range(NC) for r in range(3)]
    out = _combine(parts, dens)
    return out[:N]


# speedup vs baseline: 1.1380x; 1.0428x over previous
import functools

import jax
import jax.numpy as jnp
from jax import lax
from jax.experimental import pallas as pl
from jax.experimental.pallas import tpu as pltpu
from jax.experimental.pallas import tpu_sc as plsc

N = 10000
D = 128
E = 320000
DE = 16
NPAD = 10240

NC = 2
NS = 16
NW = NC * NS
EPW = E // NW
CB = 80
NCH = EPW // CB
LPC = CB // 16
RPT = NPAD // NS

_mesh = plsc.VectorSubcoreMesh(core_axis_name="c", subcore_axis_name="s")


def _proj_body(x_ref, wdd_ref, wdt_ref, wtt_ref,
               asdd_ref, atdd_ref, asdt_ref, atdt_ref, astt_ref, attt_ref,
               ea_ref, b_ref,
               hdd_ref, hdt_ref, htt_ref,
               sdd_ref, tdd_ref, sdt_ref, tdt_ref, stt_ref, ttt_ref,
               et_ref):
    x = x_ref[...]
    et_ref[...] = jnp.dot(ea_ref[...], b_ref[...],
                          preferred_element_type=jnp.float32)
    wrefs = (wdd_ref, wdt_ref, wtt_ref)
    arefs = (asdd_ref, atdd_ref, asdt_ref, atdt_ref, astt_ref, attt_ref)
    hrefs = (hdd_ref, hdt_ref, htt_ref)
    srefs = (sdd_ref, tdd_ref, sdt_ref, tdt_ref, stt_ref, ttt_ref)
    for r in range(3):
        h = jnp.dot(x, wrefs[r][...], preferred_element_type=jnp.float32)
        hrefs[r][...] = h
        h3 = h.reshape(8, 128, D)
        for p in range(2):
            a = arefs[2 * r + p][0, :]
            srefs[2 * r + p][...] = jnp.sum(h3 * a[None, None, :], axis=2)


def _project(x, W_dd, W_dt, W_tt, a6, ea2, B):
    blk = 1024
    eblk = E // 8 // (NPAD // blk)
    grid = (NPAD // blk,)
    wspec = pl.BlockSpec((D, D), lambda i: (0, 0))
    aspec = pl.BlockSpec((1, D), lambda i: (0, 0))
    hspec = pl.BlockSpec((blk, D), lambda i: (i, 0))
    sspec = pl.BlockSpec((8, D), lambda i: (i, 0))
    espec = pl.BlockSpec((eblk, 128), lambda i: (i, 0))
    return pl.pallas_call(
        _proj_body,
        grid=grid,
        in_specs=[hspec, wspec, wspec, wspec] + [aspec] * 6
        + [espec, pl.BlockSpec((128, 128), lambda i: (0, 0))],
        out_specs=[hspec] * 3 + [sspec] * 6 + [espec],
        out_shape=[jax.ShapeDtypeStruct((NPAD, D), jnp.float32)] * 3
        + [jax.ShapeDtypeStruct((NPAD // 128, 128), jnp.float32)] * 6
        + [jax.ShapeDtypeStruct((E // 8, 128), jnp.float32)],
    )(x, W_dd, W_dt, W_tt, *a6, ea2, B)


@functools.partial(
    pl.kernel,
    out_type=[jax.ShapeDtypeStruct((E,), jnp.float32)] * 3
    + [jax.ShapeDtypeStruct((NC, NPAD), jnp.float32)] * 3,
    mesh=_mesh,
    scratch_types=[
        pltpu.VMEM((NPAD,), jnp.float32),
        pltpu.VMEM((NPAD,), jnp.float32),
        pltpu.VMEM((EPW,), jnp.int32),
        pltpu.VMEM((NCH, CB), jnp.int32),
        pltpu.VMEM((EPW,), jnp.float32),
        pltpu.VMEM((EPW,), jnp.float32),
        pltpu.VMEM_SHARED((NPAD,), jnp.float32),
        pltpu.VMEM_SHARED((NPAD,), jnp.float32),
        pltpu.VMEM_SHARED((NPAD,), jnp.float32),
        pltpu.SemaphoreType.DMA,
    ],
    compiler_params=pltpu.CompilerParams(needs_layout_passes=False),
)
def _sc_edge_logits(s_dd, t_dd, s_dt, t_dt, s_tt, t_tt,
                    src_dd, src_dt, src_tt, dst2_dd, dst2_dt, dst2_tt,
                    eatt_hbm,
                    ex_dd, ex_dt, ex_tt, den_dd, den_dt, den_tt,
                    s_v, t_v, src_v, dst_v, ea_v, ex_v, dsh0, dsh1, dsh2,
                    dsem):
    cid = lax.axis_index("c")
    sid = lax.axis_index("s")
    wid = cid * NS + sid
    base = pl.multiple_of(wid * EPW, 8)
    rbase = pl.multiple_of(sid * RPT, 8)
    dshs = (dsh0, dsh1, dsh2)
    s_hbms = (s_dd, s_dt, s_tt)
    t_hbms = (t_dd, t_dt, t_tt)
    src_hbms = (src_dd, src_dt, src_tt)
    dst_hbms = (dst2_dd, dst2_dt, dst2_tt)
    ex_hbms = (ex_dd, ex_dt, ex_tt)
    den_hbms = (den_dd, den_dt, den_tt)

    for m in range(RPT // 16):
        ex_v[pl.ds(m * 16, 16)] = jnp.zeros((16,), jnp.float32)
    for r in range(3):
        pltpu.sync_copy(ex_v.at[pl.ds(0, RPT)], dshs[r].at[pl.ds(rbase, RPT)])
    plsc.subcore_barrier()

    for r in range(3):
        pltpu.sync_copy(s_hbms[r], s_v)
        pltpu.sync_copy(t_hbms[r], t_v)
        pltpu.sync_copy(src_hbms[r].at[pl.ds(base, EPW)], src_v)
        pltpu.sync_copy(dst_hbms[r].at[wid], dst_v)
        if r == 0:
            pltpu.sync_copy(eatt_hbm.at[pl.ds(base, EPW)], ea_v)

        def chunk_body(ch, carry, r=r):
            for k in range(LPC):
                off = ch * CB + k * 16
                si = src_v[pl.ds(off, 16)]
                ti = dst_v[ch, pl.ds(k * 16, 16)]
                e = plsc.load_gather(s_v, [si]) + plsc.load_gather(t_v, [ti])
                if r == 0:
                    e = e + ea_v[pl.ds(off, 16)]
                e = jnp.where(e >= 0.0, e, 0.2 * e)
                ex_v[pl.ds(off, 16)] = jnp.exp(e)
            pltpu.async_copy(ex_v.at[pl.ds(ch * CB, CB)],
                             dshs[r].at[dst_v.at[ch]], dsem, add=True)
            return carry

        lax.fori_loop(0, NCH, chunk_body, 0)
        pltpu.sync_copy(ex_v, ex_hbms[r].at[pl.ds(base, EPW)])

        def drain_body(ch, carry, r=r):
            pltpu.make_async_copy(ex_v.at[pl.ds(ch * CB, CB)],
                                  dshs[r].at[dst_v.at[ch]], dsem).wait()
            return carry

        lax.fori_loop(0, NCH, drain_body, 0)

    plsc.subcore_barrier()
    for r in range(3):
        pltpu.sync_copy(dshs[r].at[pl.ds(rbase, RPT)],
                        den_hbms[r].at[cid, pl.ds(rbase, RPT)])


def _den_combine_body(d0_ref, d1_ref, d2_ref, o0_ref, o1_ref, o2_ref):
    for d_ref, o_ref in ((d0_ref, o0_ref), (d1_ref, o1_ref), (d2_ref, o2_ref)):
        o_ref[...] = d_ref[0] + d_ref[1] + jnp.float32(1e-16)


def _den_combine(den3):
    ispec = pl.BlockSpec((NC, 8, 128), lambda i: (0, i, 0))
    ospec = pl.BlockSpec((8, 128), lambda i: (i, 0))
    return pl.pallas_call(
        _den_combine_body,
        grid=(NPAD // 1024,),
        in_specs=[ispec] * 3,
        out_specs=[ospec] * 3,
        out_shape=[jax.ShapeDtypeStruct((NPAD // 128, 128), jnp.float32)] * 3,
    )(*[d.reshape(NC, NPAD // 128, 128) for d in den3])


CB2 = 40
NCH2 = EPW // CB2
_GRP = ((0, 0), (16, 0), (24, 8))


@functools.partial(
    pl.kernel,
    out_type=jax.ShapeDtypeStruct((NC, 3, NPAD, D), jnp.float32),
    mesh=_mesh,
    scratch_types=[
        pltpu.VMEM((EPW,), jnp.int32),
        pltpu.VMEM((4, CB2), jnp.int32),
        pltpu.VMEM((2 * CB2,), jnp.float32),
        pltpu.VMEM((CB2, D), jnp.float32),
        pltpu.VMEM((CB2, D), jnp.float32),
        pltpu.VMEM((CB2, D), jnp.float32),
        pltpu.VMEM((CB2, D), jnp.float32),
        pltpu.SemaphoreType.DMA,
        pltpu.SemaphoreType.DMA,
        pltpu.SemaphoreType.DMA,
        pltpu.SemaphoreType.DMA,
        pltpu.VMEM_SHARED((NPAD, D), jnp.float32),
    ],
    compiler_params=pltpu.CompilerParams(needs_layout_passes=False),
)
def _sc_aggregate(h_dd, h_dt, h_tt, ex_dd, ex_dt, ex_tt,
                  src2_dd, src2_dt, src2_tt, dst2_dd, dst2_dt, dst2_tt,
                  out_hbm,
                  src_v, dst_r, att_r, g0, g1, s0, s1,
                  gsem_a, gsem_b, ssem_a, ssem_b, accum):
    cid = lax.axis_index("c")
    sid = lax.axis_index("s")
    wid = cid * NS + sid
    base = pl.multiple_of(wid * EPW, 8)
    h_hbms = (h_dd, h_dt, h_tt)
    att_hbms = (ex_dd, ex_dt, ex_tt)
    src_hbms = (src2_dd, src2_dt, src2_tt)
    dst_hbms = (dst2_dd, dst2_dt, dst2_tt)
    gbuf = (g0, g1)
    sbuf = (s0, s1)
    gsems = (gsem_a, gsem_b)
    ssems = (ssem_a, ssem_b)

    for i in range(CB2):
        for u in range(D // 16):
            s0[i, pl.ds(u * 16, 16)] = jnp.zeros((16,), jnp.float32)
    for q in range(RPT // CB2):
        st = pl.multiple_of(sid * RPT + q * CB2, 8)
        pltpu.sync_copy(s0, accum.at[pl.ds(st, CB2)])
    plsc.subcore_barrier()

    for r in range(3):
        pltpu.sync_copy(src_hbms[r].at[pl.ds(base, EPW)], src_v)

        def prefetch(ch, k4, r=r):
            gk = k4 % 2
            cbase = pl.multiple_of(base + ch * CB2, 8)
            pltpu.async_copy(att_hbms[r].at[pl.ds(cbase, CB2)],
                             att_r.at[pl.ds(gk * CB2, CB2)], gsems[gk])
            pltpu.async_copy(dst_hbms[r].at[wid * NCH2 + ch],
                             dst_r.at[k4], gsems[gk])
            pltpu.async_copy(h_hbms[r].at[src_v.at[pl.ds(ch * CB2, CB2)]],
                             gbuf[gk], gsems[gk])

        def wait_scatter(gk, k4):
            pltpu.make_async_copy(sbuf[gk], accum.at[dst_r.at[k4]],
                                  ssems[gk]).wait()

        def step(ch, k4, swait_pred, pre_ch, r=r):
            gk = k4 % 2
            cbase = pl.multiple_of(base + ch * CB2, 8)
            pltpu.make_async_copy(att_hbms[r].at[pl.ds(cbase, CB2)],
                                  att_r.at[pl.ds(gk * CB2, CB2)],
                                  gsems[gk]).wait()
            pltpu.make_async_copy(dst_hbms[r].at[wid * NCH2 + ch],
                                  dst_r.at[k4], gsems[gk]).wait()
            pltpu.make_async_copy(h_hbms[r].at[src_v.at[pl.ds(ch * CB2, CB2)]],
                                  gbuf[gk], gsems[gk]).wait()
            if swait_pred is None:
                wait_scatter(gk, (k4 + 2) % 4)
            elif swait_pred is not False:
                @pl.when(swait_pred)
                def _():
                    wait_scatter(gk, (k4 + 2) % 4)
            for off, j0 in _GRP:
                av16 = att_r[pl.ds(gk * CB2 + off, 16)]
                for j in range(j0, 16):
                    i = off + j
                    for u in range(D // 16):
                        sl = pl.ds(u * 16, 16)
                        sbuf[gk][i, sl] = gbuf[gk][i, sl] * av16[j]
            pltpu.async_copy(sbuf[gk], accum.at[dst_r.at[k4]],
                             ssems[gk], add=True)
            if pre_ch is not None:
                pre, pred = pre_ch
                if pred is None:
                    prefetch(pre, (k4 + 2) % 4)
                else:
                    @pl.when(pred)
                    def _():
                        prefetch(pre, (k4 + 2) % 4)

        prefetch(0, 0)
        prefetch(1, 1)

        def quad(q, cc):
            c0 = 4 * q
            step(c0 + 0, 0, q > 0, (c0 + 2, None))
            step(c0 + 1, 1, q > 0, (c0 + 3, None))
            step(c0 + 2, 2, None, (c0 + 4, None))
            step(c0 + 3, 3, None, (c0 + 5, None))
            return cc

        lax.fori_loop(0, (NCH2 - 2) // 4, quad, 0)
        step(NCH2 - 2, 0, None, None)
        step(NCH2 - 1, 1, None, None)
        wait_scatter(0, 0)
        wait_scatter(1, 1)

        plsc.subcore_barrier()
        for q in range(RPT // CB2):
            st = pl.multiple_of(sid * RPT + q * CB2, 8)
            pltpu.sync_copy(accum.at[pl.ds(st, CB2)],
                            out_hbm.at[cid, r, pl.ds(st, CB2)])
        if r < 2:
            for i in range(CB2):
                for u in range(D // 16):
                    s0[i, pl.ds(u * 16, 16)] = jnp.zeros((16,), jnp.float32)
            for q in range(RPT // CB2):
                st = pl.multiple_of(sid * RPT + q * CB2, 8)
                pltpu.sync_copy(s0, accum.at[pl.ds(st, CB2)])
            plsc.subcore_barrier()


def _combine_body(p00, p01, p02, p10, p11, p12, d0, d1, d2, o_ref):
    acc = (p00[...] + p10[...]) / d0[...]
    acc = acc + (p01[...] + p11[...]) / d1[...]
    acc = acc + (p02[...] + p12[...]) / d2[...]
    o_ref[...] = acc * jnp.float32(1.0 / 3.0)


def _combine(parts, dens):
    blk = 1024
    spec = pl.BlockSpec((blk, D), lambda i: (i, 0))
    return pl.pallas_call(
        _combine_body,
        grid=(NPAD // blk,),
        in_specs=[spec] * 9,
        out_specs=spec,
        out_shape=jax.ShapeDtypeStruct((NPAD, D), jnp.float32),
    )(*parts, *dens)


def kernel(all_nodes, edge_index_dd, edge_index_dt, edge_index_tt, edge_attr_dd,
           W_dd, a_src_dd, a_dst_dd, a_edge_dd,
           W_dt, a_src_dt, a_dst_dt,
           W_tt, a_src_tt, a_dst_tt):
    x = jnp.concatenate(
        [all_nodes, jnp.zeros((NPAD - N, D), jnp.float32)], axis=0)
    a6 = [a.reshape(1, D) for a in
          (a_src_dd, a_dst_dd, a_src_dt, a_dst_dt, a_src_tt, a_dst_tt)]
    ea2 = edge_attr_dd.reshape(E // 8, 8 * DE)
    rows = ((jnp.arange(8) * DE)[:, None] + jnp.arange(DE)[None, :]).reshape(-1)
    cols = jnp.repeat(jnp.arange(8), DE)
    B = jnp.zeros((8 * DE, 128), jnp.float32).at[rows, cols].set(
        jnp.tile(a_edge_dd, 8))
    (h_dd, h_dt, h_tt,
     s2_dd, t2_dd, s2_dt, t2_dt, s2_tt, t2_tt, et_pack) = _project(
        x, W_dd, W_dt, W_tt, a6, ea2, B)
    eatt = et_pack[:, :8].reshape(E)
    s_dd, t_dd = s2_dd.reshape(NPAD), t2_dd.reshape(NPAD)
    s_dt, t_dt = s2_dt.reshape(NPAD), t2_dt.reshape(NPAD)
    s_tt, t_tt = s2_tt.reshape(NPAD), t2_tt.reshape(NPAD)

    src_dd, dst_dd = edge_index_dd[0], edge_index_dd[1]
    src_dt, dst_dt = edge_index_dt[0], edge_index_dt[1]
    src_tt, dst_tt = edge_index_tt[0], edge_index_tt[1]
    d2 = lambda a: a.reshape(NW, NCH, CB)

    (ex_dd, ex_dt, ex_tt, denp_dd, denp_dt, denp_tt) = _sc_edge_logits(
        s_dd, t_dd, s_dt, t_dt, s_tt, t_tt,
        src_dd, src_dt, src_tt, d2(dst_dd), d2(dst_dt), d2(dst_tt), eatt)

    dens = [jnp.broadcast_to(d.reshape(NPAD, 1), (NPAD, D)) for d in
            _den_combine((denp_dd, denp_dt, denp_tt))]

    dflat = lambda a: a.reshape(NW * NCH2, CB2)
    out_parts = _sc_aggregate(
        h_dd, h_dt, h_tt, ex_dd, ex_dt, ex_tt,
        src_dd, src_dt, src_tt,
        dflat(dst_dd), dflat(dst_dt), dflat(dst_tt))

    parts = [out_parts[c, r] for c in range(NC) for r in range(3)]
    out = _combine(parts, dens)
    return out[:N]


# speedup vs baseline: 1.1439x; 1.0052x over previous
import functools

import jax
import jax.numpy as jnp
from jax import lax
from jax.experimental import pallas as pl
from jax.experimental.pallas import tpu as pltpu
from jax.experimental.pallas import tpu_sc as plsc

N = 10000
D = 128
E = 320000
DE = 16
NPAD = 10240

NC = 2
NS = 16
NW = NC * NS
EPW = E // NW
CB = 80
NCH = EPW // CB
LPC = CB // 16
RPT = NPAD // NS

_mesh = plsc.VectorSubcoreMesh(core_axis_name="c", subcore_axis_name="s")


def _proj_body(x_ref, wdd_ref, wdt_ref, wtt_ref,
               asdd_ref, atdd_ref, asdt_ref, atdt_ref, astt_ref, attt_ref,
               ea_ref, b_ref,
               hdd_ref, hdt_ref, htt_ref,
               sdd_ref, tdd_ref, sdt_ref, tdt_ref, stt_ref, ttt_ref,
               et_ref):
    x = x_ref[...]
    et_ref[...] = jnp.dot(ea_ref[...], b_ref[...],
                          preferred_element_type=jnp.float32)
    wrefs = (wdd_ref, wdt_ref, wtt_ref)
    arefs = (asdd_ref, atdd_ref, asdt_ref, atdt_ref, astt_ref, attt_ref)
    hrefs = (hdd_ref, hdt_ref, htt_ref)
    srefs = (sdd_ref, tdd_ref, sdt_ref, tdt_ref, stt_ref, ttt_ref)
    for r in range(3):
        h = jnp.dot(x, wrefs[r][...], preferred_element_type=jnp.float32)
        hrefs[r][...] = h
        h3 = h.reshape(8, 128, D)
        for p in range(2):
            a = arefs[2 * r + p][0, :]
            srefs[2 * r + p][...] = jnp.sum(h3 * a[None, None, :], axis=2)


def _project(x, W_dd, W_dt, W_tt, a6, ea2, B):
    blk = 1024
    eblk = E // 8 // (NPAD // blk)
    grid = (NPAD // blk,)
    wspec = pl.BlockSpec((D, D), lambda i: (0, 0))
    aspec = pl.BlockSpec((1, D), lambda i: (0, 0))
    hspec = pl.BlockSpec((blk, D), lambda i: (i, 0))
    sspec = pl.BlockSpec((8, D), lambda i: (i, 0))
    espec = pl.BlockSpec((eblk, 128), lambda i: (i, 0))
    return pl.pallas_call(
        _proj_body,
        grid=grid,
        in_specs=[hspec, wspec, wspec, wspec] + [aspec] * 6
        + [espec, pl.BlockSpec((128, 128), lambda i: (0, 0))],
        out_specs=[hspec] * 3 + [sspec] * 6 + [espec],
        out_shape=[jax.ShapeDtypeStruct((NPAD, D), jnp.float32)] * 3
        + [jax.ShapeDtypeStruct((NPAD // 128, 128), jnp.float32)] * 6
        + [jax.ShapeDtypeStruct((E // 8, 128), jnp.float32)],
    )(x, W_dd, W_dt, W_tt, *a6, ea2, B)


@functools.partial(
    pl.kernel,
    out_type=[jax.ShapeDtypeStruct((E,), jnp.float32)] * 3
    + [jax.ShapeDtypeStruct((NC, NPAD), jnp.float32)] * 3,
    mesh=_mesh,
    scratch_types=[
        pltpu.VMEM((NPAD,), jnp.float32),
        pltpu.VMEM((NPAD,), jnp.float32),
        pltpu.VMEM((EPW,), jnp.int32),
        pltpu.VMEM((NCH, CB), jnp.int32),
        pltpu.VMEM((EPW,), jnp.float32),
        pltpu.VMEM((EPW,), jnp.float32),
        pltpu.VMEM_SHARED((NPAD,), jnp.float32),
        pltpu.VMEM_SHARED((NPAD,), jnp.float32),
        pltpu.VMEM_SHARED((NPAD,), jnp.float32),
        pltpu.SemaphoreType.DMA,
    ],
    compiler_params=pltpu.CompilerParams(needs_layout_passes=False),
)
def _sc_edge_logits(s_dd, t_dd, s_dt, t_dt, s_tt, t_tt,
                    src_dd, src_dt, src_tt, dst2_dd, dst2_dt, dst2_tt,
                    eatt_hbm,
                    ex_dd, ex_dt, ex_tt, den_dd, den_dt, den_tt,
                    s_v, t_v, src_v, dst_v, ea_v, ex_v, dsh0, dsh1, dsh2,
                    dsem):
    cid = lax.axis_index("c")
    sid = lax.axis_index("s")
    wid = cid * NS + sid
    base = pl.multiple_of(wid * EPW, 8)
    rbase = pl.multiple_of(sid * RPT, 8)
    dshs = (dsh0, dsh1, dsh2)
    s_hbms = (s_dd, s_dt, s_tt)
    t_hbms = (t_dd, t_dt, t_tt)
    src_hbms = (src_dd, src_dt, src_tt)
    dst_hbms = (dst2_dd, dst2_dt, dst2_tt)
    ex_hbms = (ex_dd, ex_dt, ex_tt)
    den_hbms = (den_dd, den_dt, den_tt)

    for m in range(RPT // 16):
        ex_v[pl.ds(m * 16, 16)] = jnp.zeros((16,), jnp.float32)
    for r in range(3):
        pltpu.sync_copy(ex_v.at[pl.ds(0, RPT)], dshs[r].at[pl.ds(rbase, RPT)])
    plsc.subcore_barrier()

    for r in range(3):
        cps = [(s_hbms[r], s_v), (t_hbms[r], t_v),
               (src_hbms[r].at[pl.ds(base, EPW)], src_v),
               (dst_hbms[r].at[wid], dst_v)]
        if r == 0:
            cps.append((eatt_hbm.at[pl.ds(base, EPW)], ea_v))
        descs = [pltpu.async_copy(a, b, dsem) for a, b in cps]
        for dsc in descs:
            dsc.wait()

        def chunk_body(ch, carry, r=r):
            for k in range(LPC):
                off = ch * CB + k * 16
                si = src_v[pl.ds(off, 16)]
                ti = dst_v[ch, pl.ds(k * 16, 16)]
                e = plsc.load_gather(s_v, [si]) + plsc.load_gather(t_v, [ti])
                if r == 0:
                    e = e + ea_v[pl.ds(off, 16)]
                e = jnp.where(e >= 0.0, e, 0.2 * e)
                ex_v[pl.ds(off, 16)] = jnp.exp(e)
            pltpu.async_copy(ex_v.at[pl.ds(ch * CB, CB)],
                             dshs[r].at[dst_v.at[ch]], dsem, add=True)
            return carry

        lax.fori_loop(0, NCH, chunk_body, 0)
        pltpu.sync_copy(ex_v, ex_hbms[r].at[pl.ds(base, EPW)])

        def drain_body(ch, carry, r=r):
            pltpu.make_async_copy(ex_v.at[pl.ds(ch * CB, CB)],
                                  dshs[r].at[dst_v.at[ch]], dsem).wait()
            return carry

        lax.fori_loop(0, NCH, drain_body, 0)

    plsc.subcore_barrier()
    for r in range(3):
        pltpu.sync_copy(dshs[r].at[pl.ds(rbase, RPT)],
                        den_hbms[r].at[cid, pl.ds(rbase, RPT)])


def _den_combine_body(d0_ref, d1_ref, d2_ref, o0_ref, o1_ref, o2_ref):
    for d_ref, o_ref in ((d0_ref, o0_ref), (d1_ref, o1_ref), (d2_ref, o2_ref)):
        o_ref[...] = d_ref[0] + d_ref[1] + jnp.float32(1e-16)


def _den_combine(den3):
    ispec = pl.BlockSpec((NC, 8, 128), lambda i: (0, i, 0))
    ospec = pl.BlockSpec((8, 128), lambda i: (i, 0))
    return pl.pallas_call(
        _den_combine_body,
        grid=(NPAD // 1024,),
        in_specs=[ispec] * 3,
        out_specs=[ospec] * 3,
        out_shape=[jax.ShapeDtypeStruct((NPAD // 128, 128), jnp.float32)] * 3,
    )(*[d.reshape(NC, NPAD // 128, 128) for d in den3])


CB2 = 40
NCH2 = EPW // CB2
_GRP = ((0, 0), (16, 0), (24, 8))


@functools.partial(
    pl.kernel,
    out_type=jax.ShapeDtypeStruct((NC, 3, NPAD, D), jnp.float32),
    mesh=_mesh,
    scratch_types=[
        pltpu.VMEM((EPW,), jnp.int32),
        pltpu.VMEM((4, CB2), jnp.int32),
        pltpu.VMEM((2 * CB2,), jnp.float32),
        pltpu.VMEM((CB2, D), jnp.float32),
        pltpu.VMEM((CB2, D), jnp.float32),
        pltpu.VMEM((CB2, D), jnp.float32),
        pltpu.VMEM((CB2, D), jnp.float32),
        pltpu.SemaphoreType.DMA,
        pltpu.SemaphoreType.DMA,
        pltpu.SemaphoreType.DMA,
        pltpu.SemaphoreType.DMA,
        pltpu.VMEM_SHARED((NPAD, D), jnp.float32),
    ],
    compiler_params=pltpu.CompilerParams(needs_layout_passes=False),
)
def _sc_aggregate(h_dd, h_dt, h_tt, ex_dd, ex_dt, ex_tt,
                  src2_dd, src2_dt, src2_tt, dst2_dd, dst2_dt, dst2_tt,
                  out_hbm,
                  src_v, dst_r, att_r, g0, g1, s0, s1,
                  gsem_a, gsem_b, ssem_a, ssem_b, accum):
    cid = lax.axis_index("c")
    sid = lax.axis_index("s")
    wid = cid * NS + sid
    base = pl.multiple_of(wid * EPW, 8)
    h_hbms = (h_dd, h_dt, h_tt)
    att_hbms = (ex_dd, ex_dt, ex_tt)
    src_hbms = (src2_dd, src2_dt, src2_tt)
    dst_hbms = (dst2_dd, dst2_dt, dst2_tt)
    gbuf = (g0, g1)
    sbuf = (s0, s1)
    gsems = (gsem_a, gsem_b)
    ssems = (ssem_a, ssem_b)

    for i in range(CB2):
        for u in range(D // 16):
            s0[i, pl.ds(u * 16, 16)] = jnp.zeros((16,), jnp.float32)
    for q in range(RPT // CB2):
        st = pl.multiple_of(sid * RPT + q * CB2, 8)
        pltpu.sync_copy(s0, accum.at[pl.ds(st, CB2)])
    plsc.subcore_barrier()

    for r in range(3):
        pltpu.sync_copy(src_hbms[r].at[pl.ds(base, EPW)], src_v)

        def prefetch(ch, k4, r=r):
            gk = k4 % 2
            cbase = pl.multiple_of(base + ch * CB2, 8)
            pltpu.async_copy(att_hbms[r].at[pl.ds(cbase, CB2)],
                             att_r.at[pl.ds(gk * CB2, CB2)], gsems[gk])
            pltpu.async_copy(dst_hbms[r].at[wid * NCH2 + ch],
                             dst_r.at[k4], gsems[gk])
            pltpu.async_copy(h_hbms[r].at[src_v.at[pl.ds(ch * CB2, CB2)]],
                             gbuf[gk], gsems[gk])

        def wait_scatter(gk, k4):
            pltpu.make_async_copy(sbuf[gk], accum.at[dst_r.at[k4]],
                                  ssems[gk]).wait()

        def step(ch, k4, swait_pred, pre_ch, r=r):
            gk = k4 % 2
            cbase = pl.multiple_of(base + ch * CB2, 8)
            pltpu.make_async_copy(att_hbms[r].at[pl.ds(cbase, CB2)],
                                  att_r.at[pl.ds(gk * CB2, CB2)],
                                  gsems[gk]).wait()
            pltpu.make_async_copy(dst_hbms[r].at[wid * NCH2 + ch],
                                  dst_r.at[k4], gsems[gk]).wait()
            pltpu.make_async_copy(h_hbms[r].at[src_v.at[pl.ds(ch * CB2, CB2)]],
                                  gbuf[gk], gsems[gk]).wait()
            if swait_pred is None:
                wait_scatter(gk, (k4 + 2) % 4)
            elif swait_pred is not False:
                @pl.when(swait_pred)
                def _():
                    wait_scatter(gk, (k4 + 2) % 4)
            for off, j0 in _GRP:
                av16 = att_r[pl.ds(gk * CB2 + off, 16)]
                for j in range(j0, 16):
                    i = off + j
                    for u in range(D // 16):
                        sl = pl.ds(u * 16, 16)
                        sbuf[gk][i, sl] = gbuf[gk][i, sl] * av16[j]
            pltpu.async_copy(sbuf[gk], accum.at[dst_r.at[k4]],
                             ssems[gk], add=True)
            if pre_ch is not None:
                pre, pred = pre_ch
                if pred is None:
                    prefetch(pre, (k4 + 2) % 4)
                else:
                    @pl.when(pred)
                    def _():
                        prefetch(pre, (k4 + 2) % 4)

        prefetch(0, 0)
        prefetch(1, 1)

        def quad(q, cc):
            c0 = 4 * q
            step(c0 + 0, 0, q > 0, (c0 + 2, None))
            step(c0 + 1, 1, q > 0, (c0 + 3, None))
            step(c0 + 2, 2, None, (c0 + 4, None))
            step(c0 + 3, 3, None, (c0 + 5, None))
            return cc

        lax.fori_loop(0, (NCH2 - 2) // 4, quad, 0)
        step(NCH2 - 2, 0, None, None)
        step(NCH2 - 1, 1, None, None)
        wait_scatter(0, 0)
        wait_scatter(1, 1)

        plsc.subcore_barrier()
        for q in range(RPT // CB2):
            st = pl.multiple_of(sid * RPT + q * CB2, 8)
            pltpu.sync_copy(accum.at[pl.ds(st, CB2)],
                            out_hbm.at[cid, r, pl.ds(st, CB2)])
        if r < 2:
            for i in range(CB2):
                for u in range(D // 16):
                    s0[i, pl.ds(u * 16, 16)] = jnp.zeros((16,), jnp.float32)
            for q in range(RPT // CB2):
                st = pl.multiple_of(sid * RPT + q * CB2, 8)
                pltpu.sync_copy(s0, accum.at[pl.ds(st, CB2)])
            plsc.subcore_barrier()


def _combine_body(p00, p01, p02, p10, p11, p12, d0, d1, d2, o_ref):
    acc = (p00[...] + p10[...]) / d0[...]
    acc = acc + (p01[...] + p11[...]) / d1[...]
    acc = acc + (p02[...] + p12[...]) / d2[...]
    o_ref[...] = acc * jnp.float32(1.0 / 3.0)


def _combine(parts, dens):
    blk = 1024
    spec = pl.BlockSpec((blk, D), lambda i: (i, 0))
    return pl.pallas_call(
        _combine_body,
        grid=(NPAD // blk,),
        in_specs=[spec] * 9,
        out_specs=spec,
        out_shape=jax.ShapeDtypeStruct((NPAD, D), jnp.float32),
    )(*parts, *dens)


def kernel(all_nodes, edge_index_dd, edge_index_dt, edge_index_tt, edge_attr_dd,
           W_dd, a_src_dd, a_dst_dd, a_edge_dd,
           W_dt, a_src_dt, a_dst_dt,
           W_tt, a_src_tt, a_dst_tt):
    x = jnp.concatenate(
        [all_nodes, jnp.zeros((NPAD - N, D), jnp.float32)], axis=0)
    a6 = [a.reshape(1, D) for a in
          (a_src_dd, a_dst_dd, a_src_dt, a_dst_dt, a_src_tt, a_dst_tt)]
    ea2 = edge_attr_dd.reshape(E // 8, 8 * DE)
    rows = ((jnp.arange(8) * DE)[:, None] + jnp.arange(DE)[None, :]).reshape(-1)
    cols = jnp.repeat(jnp.arange(8), DE)
    B = jnp.zeros((8 * DE, 128), jnp.float32).at[rows, cols].set(
        jnp.tile(a_edge_dd, 8))
    (h_dd, h_dt, h_tt,
     s2_dd, t2_dd, s2_dt, t2_dt, s2_tt, t2_tt, et_pack) = _project(
        x, W_dd, W_dt, W_tt, a6, ea2, B)
    eatt = et_pack[:, :8].reshape(E)
    s_dd, t_dd = s2_dd.reshape(NPAD), t2_dd.reshape(NPAD)
    s_dt, t_dt = s2_dt.reshape(NPAD), t2_dt.reshape(NPAD)
    s_tt, t_tt = s2_tt.reshape(NPAD), t2_tt.reshape(NPAD)

    src_dd, dst_dd = edge_index_dd[0], edge_index_dd[1]
    src_dt, dst_dt = edge_index_dt[0], edge_index_dt[1]
    src_tt, dst_tt = edge_index_tt[0], edge_index_tt[1]
    d2 = lambda a: a.reshape(NW, NCH, CB)

    (ex_dd, ex_dt, ex_tt, denp_dd, denp_dt, denp_tt) = _sc_edge_logits(
        s_dd, t_dd, s_dt, t_dt, s_tt, t_tt,
        src_dd, src_dt, src_tt, d2(dst_dd), d2(dst_dt), d2(dst_tt), eatt)

    dens = [jnp.broadcast_to(d.reshape(NPAD, 1), (NPAD, D)) for d in
            _den_combine((denp_dd, denp_dt, denp_tt))]

    dflat = lambda a: a.reshape(NW * NCH2, CB2)
    out_parts = _sc_aggregate(
        h_dd, h_dt, h_tt, ex_dd, ex_dt, ex_tt,
        src_dd, src_dt, src_tt,
        dflat(dst_dd), dflat(dst_dt), dflat(dst_tt))

    parts = [out_parts[c, r] for c in range(NC) for r in range(3)]
    out = _combine(parts, dens)
    return out[:N]


# speedup vs baseline: 1.1503x; 1.0056x over previous
import functools

import jax
import jax.numpy as jnp
from jax import lax
from jax.experimental import pallas as pl
from jax.experimental.pallas import tpu as pltpu
from jax.experimental.pallas import tpu_sc as plsc

N = 10000
D = 128
E = 320000
DE = 16
NPAD = 10240

NC = 2
NS = 16
NW = NC * NS
EPW = E // NW
CB = 80
NCH = EPW // CB
LPC = CB // 16
RPT = NPAD // NS

_mesh = plsc.VectorSubcoreMesh(core_axis_name="c", subcore_axis_name="s")


def _proj_body(x_ref, wdd_ref, wdt_ref, wtt_ref,
               asdd_ref, atdd_ref, asdt_ref, atdt_ref, astt_ref, attt_ref,
               ea_ref, b_ref,
               hdd_ref, hdt_ref, htt_ref,
               sdd_ref, tdd_ref, sdt_ref, tdt_ref, stt_ref, ttt_ref,
               et_ref):
    x = x_ref[...]
    et_ref[...] = jnp.dot(ea_ref[...], b_ref[...],
                          preferred_element_type=jnp.float32)
    wrefs = (wdd_ref, wdt_ref, wtt_ref)
    arefs = (asdd_ref, atdd_ref, asdt_ref, atdt_ref, astt_ref, attt_ref)
    hrefs = (hdd_ref, hdt_ref, htt_ref)
    srefs = (sdd_ref, tdd_ref, sdt_ref, tdt_ref, stt_ref, ttt_ref)
    for r in range(3):
        h = jnp.dot(x, wrefs[r][...], preferred_element_type=jnp.float32)
        hrefs[r][...] = h
        h3 = h.reshape(8, 128, D)
        for p in range(2):
            a = arefs[2 * r + p][0, :]
            srefs[2 * r + p][...] = jnp.sum(h3 * a[None, None, :], axis=2)


def _project(x, W_dd, W_dt, W_tt, a6, ea2, B):
    blk = 1024
    eblk = E // 8 // (NPAD // blk)
    grid = (NPAD // blk,)
    wspec = pl.BlockSpec((D, D), lambda i: (0, 0))
    aspec = pl.BlockSpec((1, D), lambda i: (0, 0))
    hspec = pl.BlockSpec((blk, D), lambda i: (i, 0))
    sspec = pl.BlockSpec((8, D), lambda i: (i, 0))
    espec = pl.BlockSpec((eblk, 128), lambda i: (i, 0))
    return pl.pallas_call(
        _proj_body,
        grid=grid,
        in_specs=[hspec, wspec, wspec, wspec] + [aspec] * 6
        + [espec, pl.BlockSpec((128, 128), lambda i: (0, 0))],
        out_specs=[hspec] * 3 + [sspec] * 6 + [espec],
        out_shape=[jax.ShapeDtypeStruct((NPAD, D), jnp.float32)] * 3
        + [jax.ShapeDtypeStruct((NPAD // 128, 128), jnp.float32)] * 6
        + [jax.ShapeDtypeStruct((E // 8, 128), jnp.float32)],
    )(x, W_dd, W_dt, W_tt, *a6, ea2, B)


@functools.partial(
    pl.kernel,
    out_type=[jax.ShapeDtypeStruct((E,), jnp.float32)] * 3
    + [jax.ShapeDtypeStruct((NC, NPAD), jnp.float32)] * 3,
    mesh=_mesh,
    scratch_types=[
        pltpu.VMEM((NPAD,), jnp.float32),
        pltpu.VMEM((NPAD,), jnp.float32),
        pltpu.VMEM((EPW,), jnp.int32),
        pltpu.VMEM((NCH, CB), jnp.int32),
        pltpu.VMEM((EPW,), jnp.float32),
        pltpu.VMEM((EPW,), jnp.float32),
        pltpu.VMEM_SHARED((NPAD,), jnp.float32),
        pltpu.VMEM_SHARED((NPAD,), jnp.float32),
        pltpu.VMEM_SHARED((NPAD,), jnp.float32),
        pltpu.SemaphoreType.DMA,
    ],
    compiler_params=pltpu.CompilerParams(needs_layout_passes=False),
)
def _sc_edge_logits(s_dd, t_dd, s_dt, t_dt, s_tt, t_tt,
                    src_dd, src_dt, src_tt, dst2_dd, dst2_dt, dst2_tt,
                    eatt_hbm,
                    ex_dd, ex_dt, ex_tt, den_dd, den_dt, den_tt,
                    s_v, t_v, src_v, dst_v, ea_v, ex_v, dsh0, dsh1, dsh2,
                    dsem):
    cid = lax.axis_index("c")
    sid = lax.axis_index("s")
    wid = cid * NS + sid
    base = pl.multiple_of(wid * EPW, 8)
    rbase = pl.multiple_of(sid * RPT, 8)
    dshs = (dsh0, dsh1, dsh2)
    s_hbms = (s_dd, s_dt, s_tt)
    t_hbms = (t_dd, t_dt, t_tt)
    src_hbms = (src_dd, src_dt, src_tt)
    dst_hbms = (dst2_dd, dst2_dt, dst2_tt)
    ex_hbms = (ex_dd, ex_dt, ex_tt)
    den_hbms = (den_dd, den_dt, den_tt)

    for m in range(RPT // 16):
        ex_v[pl.ds(m * 16, 16)] = jnp.zeros((16,), jnp.float32)
    for r in range(3):
        pltpu.sync_copy(ex_v.at[pl.ds(0, RPT)], dshs[r].at[pl.ds(rbase, RPT)])
    plsc.subcore_barrier()

    for r in range(3):
        cps = [(s_hbms[r], s_v), (t_hbms[r], t_v),
               (src_hbms[r].at[pl.ds(base, EPW)], src_v),
               (dst_hbms[r].at[wid], dst_v)]
        if r == 0:
            cps.append((eatt_hbm.at[pl.ds(base, EPW)], ea_v))
        descs = [pltpu.async_copy(a, b, dsem) for a, b in cps]
        for dsc in descs:
            dsc.wait()

        def chunk_body(ch, carry, r=r):
            for k in range(LPC):
                off = ch * CB + k * 16
                si = src_v[pl.ds(off, 16)]
                ti = dst_v[ch, pl.ds(k * 16, 16)]
                e = plsc.load_gather(s_v, [si]) + plsc.load_gather(t_v, [ti])
                if r == 0:
                    e = e + ea_v[pl.ds(off, 16)]
                e = jnp.where(e >= 0.0, e, 0.2 * e)
                ex_v[pl.ds(off, 16)] = jnp.exp(e)
            pltpu.async_copy(ex_v.at[pl.ds(ch * CB, CB)],
                             dshs[r].at[dst_v.at[ch]], dsem, add=True)
            return carry

        lax.fori_loop(0, NCH, chunk_body, 0)
        pltpu.sync_copy(ex_v, ex_hbms[r].at[pl.ds(base, EPW)])

        def drain_body(ch, carry, r=r):
            pltpu.make_async_copy(ex_v.at[pl.ds(ch * CB, CB)],
                                  dshs[r].at[dst_v.at[ch]], dsem).wait()
            return carry

        lax.fori_loop(0, NCH, drain_body, 0)

    plsc.subcore_barrier()
    for r in range(3):
        pltpu.sync_copy(dshs[r].at[pl.ds(rbase, RPT)],
                        den_hbms[r].at[cid, pl.ds(rbase, RPT)])


def _den_combine_body(d0_ref, d1_ref, d2_ref, o0_ref, o1_ref, o2_ref):
    for d_ref, o_ref in ((d0_ref, o0_ref), (d1_ref, o1_ref), (d2_ref, o2_ref)):
        o_ref[...] = d_ref[0] + d_ref[1] + jnp.float32(1e-16)


def _den_combine(den3):
    ispec = pl.BlockSpec((NC, 8, 128), lambda i: (0, i, 0))
    ospec = pl.BlockSpec((8, 128), lambda i: (i, 0))
    return pl.pallas_call(
        _den_combine_body,
        grid=(NPAD // 1024,),
        in_specs=[ispec] * 3,
        out_specs=[ospec] * 3,
        out_shape=[jax.ShapeDtypeStruct((NPAD // 128, 128), jnp.float32)] * 3,
    )(*[d.reshape(NC, NPAD // 128, 128) for d in den3])


CB2 = 40
NCH2 = EPW // CB2
_GRP = ((0, 0), (16, 0), (24, 8))


@functools.partial(
    pl.kernel,
    out_type=jax.ShapeDtypeStruct((NC, 3, NPAD, D), jnp.float32),
    mesh=_mesh,
    scratch_types=[
        pltpu.VMEM((EPW,), jnp.int32),
        pltpu.VMEM((EPW,), jnp.int32),
        pltpu.VMEM((4, CB2), jnp.int32),
        pltpu.VMEM((2 * CB2,), jnp.float32),
        pltpu.VMEM((CB2, D), jnp.float32),
        pltpu.VMEM((CB2, D), jnp.float32),
        pltpu.VMEM((CB2, D), jnp.float32),
        pltpu.VMEM((CB2, D), jnp.float32),
        pltpu.SemaphoreType.DMA,
        pltpu.SemaphoreType.DMA,
        pltpu.SemaphoreType.DMA,
        pltpu.SemaphoreType.DMA,
        pltpu.SemaphoreType.DMA,
        pltpu.VMEM_SHARED((NPAD, D), jnp.float32),
    ],
    compiler_params=pltpu.CompilerParams(needs_layout_passes=False),
)
def _sc_aggregate(h_dd, h_dt, h_tt, ex_dd, ex_dt, ex_tt,
                  src2_dd, src2_dt, src2_tt, dst2_dd, dst2_dt, dst2_tt,
                  out_hbm,
                  src_a, src_b, dst_r, att_r, g0, g1, s0, s1,
                  gsem_a, gsem_b, ssem_a, ssem_b, srcsem, accum):
    cid = lax.axis_index("c")
    sid = lax.axis_index("s")
    wid = cid * NS + sid
    base = pl.multiple_of(wid * EPW, 8)
    h_hbms = (h_dd, h_dt, h_tt)
    att_hbms = (ex_dd, ex_dt, ex_tt)
    src_hbms = (src2_dd, src2_dt, src2_tt)
    dst_hbms = (dst2_dd, dst2_dt, dst2_tt)
    gbuf = (g0, g1)
    sbuf = (s0, s1)
    gsems = (gsem_a, gsem_b)
    ssems = (ssem_a, ssem_b)
    srcbufs = (src_a, src_b)
    pltpu.async_copy(src_hbms[0].at[pl.ds(base, EPW)], src_a, srcsem)

    for i in range(CB2):
        for u in range(D // 16):
            s0[i, pl.ds(u * 16, 16)] = jnp.zeros((16,), jnp.float32)
    for q in range(RPT // CB2):
        st = pl.multiple_of(sid * RPT + q * CB2, 8)
        pltpu.sync_copy(s0, accum.at[pl.ds(st, CB2)])
    plsc.subcore_barrier()

    for r in range(3):
        src_v = srcbufs[r % 2]
        pltpu.make_async_copy(src_hbms[r].at[pl.ds(base, EPW)], src_v,
                              srcsem).wait()
        if r < 2:
            pltpu.async_copy(src_hbms[r + 1].at[pl.ds(base, EPW)],
                             srcbufs[(r + 1) % 2], srcsem)

        def prefetch(ch, k4, src_v=src_v, r=r):
            gk = k4 % 2
            cbase = pl.multiple_of(base + ch * CB2, 8)
            pltpu.async_copy(att_hbms[r].at[pl.ds(cbase, CB2)],
                             att_r.at[pl.ds(gk * CB2, CB2)], gsems[gk])
            pltpu.async_copy(dst_hbms[r].at[wid * NCH2 + ch],
                             dst_r.at[k4], gsems[gk])
            pltpu.async_copy(h_hbms[r].at[src_v.at[pl.ds(ch * CB2, CB2)]],
                             gbuf[gk], gsems[gk])

        def wait_scatter(gk, k4):
            pltpu.make_async_copy(sbuf[gk], accum.at[dst_r.at[k4]],
                                  ssems[gk]).wait()

        def step(ch, k4, swait_pred, pre_ch, src_v=src_v, r=r):
            gk = k4 % 2
            cbase = pl.multiple_of(base + ch * CB2, 8)
            pltpu.make_async_copy(att_hbms[r].at[pl.ds(cbase, CB2)],
                                  att_r.at[pl.ds(gk * CB2, CB2)],
                                  gsems[gk]).wait()
            pltpu.make_async_copy(dst_hbms[r].at[wid * NCH2 + ch],
                                  dst_r.at[k4], gsems[gk]).wait()
            pltpu.make_async_copy(h_hbms[r].at[src_v.at[pl.ds(ch * CB2, CB2)]],
                                  gbuf[gk], gsems[gk]).wait()
            if swait_pred is None:
                wait_scatter(gk, (k4 + 2) % 4)
            elif swait_pred is not False:
                @pl.when(swait_pred)
                def _():
                    wait_scatter(gk, (k4 + 2) % 4)
            for off, j0 in _GRP:
                av16 = att_r[pl.ds(gk * CB2 + off, 16)]
                for j in range(j0, 16):
                    i = off + j
                    for u in range(D // 16):
                        sl = pl.ds(u * 16, 16)
                        sbuf[gk][i, sl] = gbuf[gk][i, sl] * av16[j]
            pltpu.async_copy(sbuf[gk], accum.at[dst_r.at[k4]],
                             ssems[gk], add=True)
            if pre_ch is not None:
                pre, pred = pre_ch
                if pred is None:
                    prefetch(pre, (k4 + 2) % 4)
                else:
                    @pl.when(pred)
                    def _():
                        prefetch(pre, (k4 + 2) % 4)

        prefetch(0, 0)
        prefetch(1, 1)

        def quad(q, cc):
            c0 = 4 * q
            step(c0 + 0, 0, q > 0, (c0 + 2, None))
            step(c0 + 1, 1, q > 0, (c0 + 3, None))
            step(c0 + 2, 2, None, (c0 + 4, None))
            step(c0 + 3, 3, None, (c0 + 5, None))
            return cc

        lax.fori_loop(0, (NCH2 - 2) // 4, quad, 0)
        step(NCH2 - 2, 0, None, None)
        step(NCH2 - 1, 1, None, None)
        wait_scatter(0, 0)
        wait_scatter(1, 1)

        plsc.subcore_barrier()
        for q in range(RPT // CB2):
            st = pl.multiple_of(sid * RPT + q * CB2, 8)
            pltpu.sync_copy(accum.at[pl.ds(st, CB2)],
                            out_hbm.at[cid, r, pl.ds(st, CB2)])
        if r < 2:
            for i in range(CB2):
                for u in range(D // 16):
                    s0[i, pl.ds(u * 16, 16)] = jnp.zeros((16,), jnp.float32)
            for q in range(RPT // CB2):
                st = pl.multiple_of(sid * RPT + q * CB2, 8)
                pltpu.sync_copy(s0, accum.at[pl.ds(st, CB2)])
            plsc.subcore_barrier()


def _combine_body(p00, p01, p02, p10, p11, p12, d0, d1, d2, o_ref):
    acc = (p00[...] + p10[...]) / d0[...]
    acc = acc + (p01[...] + p11[...]) / d1[...]
    acc = acc + (p02[...] + p12[...]) / d2[...]
    o_ref[...] = acc * jnp.float32(1.0 / 3.0)


def _combine(parts, dens):
    blk = 1024
    spec = pl.BlockSpec((blk, D), lambda i: (i, 0))
    return pl.pallas_call(
        _combine_body,
        grid=(NPAD // blk,),
        in_specs=[spec] * 9,
        out_specs=spec,
        out_shape=jax.ShapeDtypeStruct((NPAD, D), jnp.float32),
    )(*parts, *dens)


def kernel(all_nodes, edge_index_dd, edge_index_dt, edge_index_tt, edge_attr_dd,
           W_dd, a_src_dd, a_dst_dd, a_edge_dd,
           W_dt, a_src_dt, a_dst_dt,
           W_tt, a_src_tt, a_dst_tt):
    x = jnp.concatenate(
        [all_nodes, jnp.zeros((NPAD - N, D), jnp.float32)], axis=0)
    a6 = [a.reshape(1, D) for a in
          (a_src_dd, a_dst_dd, a_src_dt, a_dst_dt, a_src_tt, a_dst_tt)]
    ea2 = edge_attr_dd.reshape(E // 8, 8 * DE)
    rows = ((jnp.arange(8) * DE)[:, None] + jnp.arange(DE)[None, :]).reshape(-1)
    cols = jnp.repeat(jnp.arange(8), DE)
    B = jnp.zeros((8 * DE, 128), jnp.float32).at[rows, cols].set(
        jnp.tile(a_edge_dd, 8))
    (h_dd, h_dt, h_tt,
     s2_dd, t2_dd, s2_dt, t2_dt, s2_tt, t2_tt, et_pack) = _project(
        x, W_dd, W_dt, W_tt, a6, ea2, B)
    eatt = et_pack[:, :8].reshape(E)
    s_dd, t_dd = s2_dd.reshape(NPAD), t2_dd.reshape(NPAD)
    s_dt, t_dt = s2_dt.reshape(NPAD), t2_dt.reshape(NPAD)
    s_tt, t_tt = s2_tt.reshape(NPAD), t2_tt.reshape(NPAD)

    src_dd, dst_dd = edge_index_dd[0], edge_index_dd[1]
    src_dt, dst_dt = edge_index_dt[0], edge_index_dt[1]
    src_tt, dst_tt = edge_index_tt[0], edge_index_tt[1]
    d2 = lambda a: a.reshape(NW, NCH, CB)

    (ex_dd, ex_dt, ex_tt, denp_dd, denp_dt, denp_tt) = _sc_edge_logits(
        s_dd, t_dd, s_dt, t_dt, s_tt, t_tt,
        src_dd, src_dt, src_tt, d2(dst_dd), d2(dst_dt), d2(dst_tt), eatt)

    dens = [jnp.broadcast_to(d.reshape(NPAD, 1), (NPAD, D)) for d in
            _den_combine((denp_dd, denp_dt, denp_tt))]

    dflat = lambda a: a.reshape(NW * NCH2, CB2)
    out_parts = _sc_aggregate(
        h_dd, h_dt, h_tt, ex_dd, ex_dt, ex_tt,
        src_dd, src_dt, src_tt,
        dflat(dst_dd), dflat(dst_dt), dflat(dst_tt))

    parts = [out_parts[c, r] for c in range(NC) for r in range(3)]
    out = _combine(parts, dens)
    return out[:N]
